# Initial kernel scaffold; baseline (speedup 1.0000x reference)
import jax
import jax.numpy as jnp
from jax.experimental import pallas as pl


def kernel(tokens, w0, w1):
    raise NotImplementedError("write your pallas kernel here")


# speedup vs baseline: 16.3576x; 16.3576x over previous
import jax
import jax.numpy as jnp
from jax.experimental import pallas as pl
from jax.experimental.pallas import tpu as pltpu

_BATCH = 16384
_HIST = 200
_N = _BATCH * _HIST
_COLS = 256
_ROWS = _N // _COLS
_BB = 512
_GRID = _ROWS // _BB


_CHUNK = 8


def _body(w0_ref, w1_ref, tok_ref, out_ref):
    w0 = w0_ref[0]
    w1 = w1_ref[0]

    parity = jax.lax.broadcasted_iota(jnp.int32, (_CHUNK, 2 * _COLS), 1) % 2

    def step(i, _):
        t = tok_ref[pl.ds(i * _CHUNK, _CHUNK), :].astype(jnp.float32)
        td = jnp.repeat(t, 2, axis=1)
        out_ref[pl.ds(i * _CHUNK, _CHUNK), :] = jnp.where(
            parity == 0, w0 - w1 * td * td, -td
        )
        return 0

    jax.lax.fori_loop(0, _BB // _CHUNK, step, 0)


def kernel(tokens, w0, w1):
    tok2d = tokens.reshape(_ROWS, _COLS)
    out2d = pl.pallas_call(
        _body,
        grid=(_GRID,),
        in_specs=[
            pl.BlockSpec(memory_space=pltpu.SMEM),
            pl.BlockSpec(memory_space=pltpu.SMEM),
            pl.BlockSpec((_BB, _COLS), lambda i: (i, 0)),
        ],
        out_specs=pl.BlockSpec((_BB, 2 * _COLS), lambda i: (i, 0)),
        out_shape=jax.ShapeDtypeStruct((_ROWS, 2 * _COLS), jnp.float32),
        compiler_params=pltpu.CompilerParams(
            dimension_semantics=("parallel",),
        ),
    )(w0.reshape(1), w1.reshape(1), tok2d)
    return out2d.reshape(_BATCH, _HIST, 2)


# speedup vs baseline: 788.9131x; 48.2292x over previous
import jax
import jax.numpy as jnp
from jax.experimental import pallas as pl
from jax.experimental.pallas import tpu as pltpu

_BATCH = 16384
_HIST = 200
_BT = _BATCH // 128
_K1_ROWS = 1024
_K1_GRID = _BATCH // _K1_ROWS
_K2_H = 8
_K2_GRID = _HIST // _K2_H


def _k1_body(tok_ref, t3_ref):
    for bc in range(_K1_ROWS // 128):
        src = tok_ref[pl.ds(bc * 128, 128), :]
        lo = src[:, :128].T
        hi = src[:, 128:].T
        t3_ref[pl.ds(0, 128), bc, :] = lo
        t3_ref[pl.ds(128, 72), bc, :] = hi[:72, :]


def _k2_body(w0_ref, w1_ref, t3_ref, y_ref):
    w0 = w0_ref[0]
    w1 = w1_ref[0]

    for h in range(_K2_H):
        t = t3_ref[h].astype(jnp.float32)
        a = w0 - w1 * t * t
        b = -t
        y_ref[pl.ds(h * 256, 256), :] = jnp.stack([a, b], axis=1).reshape(256, 128)


def kernel(tokens, w0, w1):
    t3 = pl.pallas_call(
        _k1_body,
        grid=(_K1_GRID,),
        in_specs=[pl.BlockSpec((_K1_ROWS, 256), lambda i: (i, 0))],
        out_specs=pl.BlockSpec((_HIST, _K1_ROWS // 128, 128), lambda i: (0, i, 0)),
        out_shape=jax.ShapeDtypeStruct((_HIST, _BT, 128), jnp.int32),
        compiler_params=pltpu.CompilerParams(
            dimension_semantics=("arbitrary",),
        ),
    )(tokens)
    y = pl.pallas_call(
        _k2_body,
        grid=(_K2_GRID,),
        in_specs=[
            pl.BlockSpec(memory_space=pltpu.SMEM),
            pl.BlockSpec(memory_space=pltpu.SMEM),
            pl.BlockSpec((_K2_H, _BT, 128), lambda i: (i, 0, 0)),
        ],
        out_specs=pl.BlockSpec((_K2_H * 256, 128), lambda i: (i, 0)),
        out_shape=jax.ShapeDtypeStruct((2 * _HIST * _BT, 128), jnp.float32),
        compiler_params=pltpu.CompilerParams(
            dimension_semantics=("arbitrary",),
        ),
    )(w0.reshape(1), w1.reshape(1), t3)
    y4 = y.reshape(_HIST, _BT, 2, 128)
    return jnp.transpose(y4, (1, 3, 0, 2)).reshape(_BATCH, _HIST, 2)


# speedup vs baseline: 935.0807x; 1.1853x over previous
import jax
import jax.numpy as jnp
from jax.experimental import pallas as pl
from jax.experimental.pallas import tpu as pltpu

_BATCH = 16384
_HIST = 200
_BT = _BATCH // 128
_KH = 8
_GRID = _HIST // _KH


def _body(w0_ref, w1_ref, tok_ref, y_ref, t3_ref):
    i = pl.program_id(0)

    @pl.when(i == 0)
    def _transpose_all():
        for bc in range(_BT):
            src = tok_ref[pl.ds(bc * 128, 128), :]
            t3_ref[pl.ds(0, 128), bc, :] = src[:, :128].T
            t3_ref[pl.ds(128, 72), bc, :] = src[:, 128:].T[:72, :]

    w0 = w0_ref[0]
    w1 = w1_ref[0]
    h0 = i * _KH
    for h in range(_KH):
        t = t3_ref[pl.ds(h0 + h, 1)][0].astype(jnp.float32)
        a = w0 - w1 * t * t
        b = -t
        y_ref[pl.ds(h * 256, 256), :] = jnp.stack([a, b], axis=1).reshape(256, 128)


def kernel(tokens, w0, w1):
    y = pl.pallas_call(
        _body,
        grid=(_GRID,),
        in_specs=[
            pl.BlockSpec(memory_space=pltpu.SMEM),
            pl.BlockSpec(memory_space=pltpu.SMEM),
            pl.BlockSpec((_BATCH, _HIST), lambda i: (0, 0)),
        ],
        out_specs=pl.BlockSpec((_KH * 256, 128), lambda i: (i, 0)),
        out_shape=jax.ShapeDtypeStruct((2 * _HIST * _BT, 128), jnp.float32),
        scratch_shapes=[pltpu.VMEM((_HIST, _BT, 128), jnp.int32)],
        compiler_params=pltpu.CompilerParams(
            dimension_semantics=("arbitrary",),
        ),
    )(w0.reshape(1), w1.reshape(1), tokens)
    y4 = y.reshape(_HIST, _BT, 2, 128)
    return jnp.transpose(y4, (1, 3, 0, 2)).reshape(_BATCH, _HIST, 2)


# speedup vs baseline: 1038.0034x; 1.1101x over previous
import jax
import jax.numpy as jnp
from jax.experimental import pallas as pl
from jax.experimental.pallas import tpu as pltpu

_BATCH = 16384
_HIST = 200
_BT = _BATCH // 128
_KH = 8
_GRID = _HIST // _KH
_HSPLIT = 16


def _body(w0_ref, w1_ref, tok_ref, y_ref, td_ref):
    i = pl.program_id(0)

    @pl.when(i == 0)
    def _transpose_lo():
        for bc in range(_BT):
            src = tok_ref[pl.ds(bc * 128, 128), :]
            lof = src[:, :128].T.astype(jnp.float32)
            td_ref[pl.ds(0, 128), 2 * bc, :] = lof
            td_ref[pl.ds(0, 128), 2 * bc + 1, :] = lof

    @pl.when(i == _HSPLIT)
    def _transpose_hi():
        for bc in range(_BT):
            src = tok_ref[pl.ds(bc * 128, 128), :]
            hif = src[:, 128:].T[:72, :].astype(jnp.float32)
            td_ref[pl.ds(0, 72), 2 * bc, :] = hif
            td_ref[pl.ds(0, 72), 2 * bc + 1, :] = hif

    w0 = w0_ref[0]
    w1 = w1_ref[0]
    even = (jax.lax.broadcasted_iota(jnp.int32, (256, 128), 0) & 1) == 0
    r0 = i * _KH - jnp.where(i >= _HSPLIT, 128, 0)
    for h in range(_KH):
        u = td_ref[pl.ds(r0 + h, 1)][0]
        y_ref[pl.ds(h * 256, 256), :] = jnp.where(even, w0 - w1 * u * u, -u)


def kernel(tokens, w0, w1):
    y = pl.pallas_call(
        _body,
        grid=(_GRID,),
        in_specs=[
            pl.BlockSpec(memory_space=pltpu.SMEM),
            pl.BlockSpec(memory_space=pltpu.SMEM),
            pl.BlockSpec((_BATCH, _HIST), lambda i: (0, 0)),
        ],
        out_specs=pl.BlockSpec((_KH * 256, 128), lambda i: (i, 0)),
        out_shape=jax.ShapeDtypeStruct((2 * _HIST * _BT, 128), jnp.float32),
        scratch_shapes=[pltpu.VMEM((128, 2 * _BT, 128), jnp.float32)],
        compiler_params=pltpu.CompilerParams(
            dimension_semantics=("arbitrary",),
        ),
    )(w0.reshape(1), w1.reshape(1), tokens)
    y4 = y.reshape(_HIST, _BT, 2, 128)
    return jnp.transpose(y4, (1, 3, 0, 2)).reshape(_BATCH, _HIST, 2)


# speedup vs baseline: 1055.9764x; 1.0173x over previous
import jax
import jax.numpy as jnp
from jax.experimental import pallas as pl
from jax.experimental.pallas import tpu as pltpu

_BATCH = 16384
_HIST = 200
_BT = _BATCH // 128
_KH = 8
_NCOMP = _HIST // _KH
_CHUNK = 1024
_NTR = _BATCH // _CHUNK
_GRID = _NTR + _NCOMP


def _body(w0_ref, w1_ref, tok_ref, y_ref, td_ref):
    i = pl.program_id(0)

    @pl.when(i < _NTR)
    def _transpose_chunk():
        for bcl in range(_CHUNK // 128):
            src = tok_ref[pl.ds(bcl * 128, 128), :]
            lof = src[:, :128].T.astype(jnp.float32)
            hif = src[:, 128:].T[:72, :].astype(jnp.float32)
            col = 2 * (i * (_CHUNK // 128) + bcl)
            for d in range(2):
                td_ref[pl.ds(0, 128), pl.ds(col + d, 1), :] = lof[:, None, :]
                td_ref[pl.ds(128, 72), pl.ds(col + d, 1), :] = hif[:, None, :]

    @pl.when(i >= _NTR)
    def _compute():
        w0 = w0_ref[0]
        w1 = w1_ref[0]
        even = (jax.lax.broadcasted_iota(jnp.int32, (256, 128), 0) & 1) == 0
        r0 = (i - _NTR) * _KH
        for h in range(_KH):
            u = td_ref[pl.ds(r0 + h, 1)][0]
            y_ref[pl.ds(h * 256, 256), :] = jnp.where(even, w0 - w1 * u * u, -u)


def kernel(tokens, w0, w1):
    y = pl.pallas_call(
        _body,
        grid=(_GRID,),
        in_specs=[
            pl.BlockSpec(memory_space=pltpu.SMEM),
            pl.BlockSpec(memory_space=pltpu.SMEM),
            pl.BlockSpec((_CHUNK, _HIST), lambda i: (jnp.minimum(i, _NTR - 1), 0)),
        ],
        out_specs=pl.BlockSpec(
            (_KH * 256, 128), lambda i: (jnp.maximum(i - _NTR, 0), 0)
        ),
        out_shape=jax.ShapeDtypeStruct((2 * _HIST * _BT, 128), jnp.float32),
        scratch_shapes=[pltpu.VMEM((_HIST, 2 * _BT, 128), jnp.float32)],
        compiler_params=pltpu.CompilerParams(
            dimension_semantics=("arbitrary",),
        ),
    )(w0.reshape(1), w1.reshape(1), tokens)
    y4 = y.reshape(_HIST, _BT, 2, 128)
    return jnp.transpose(y4, (1, 3, 0, 2)).reshape(_BATCH, _HIST, 2)


# speedup vs baseline: 1080.0608x; 1.0228x over previous
import jax
import jax.numpy as jnp
from jax.experimental import pallas as pl
from jax.experimental.pallas import tpu as pltpu

_BATCH = 16384
_HIST = 200
_BT = _BATCH // 128
_KH = 8
_NCOMP = _HIST // _KH
_CHUNK = 1024
_NTR = _BATCH // _CHUNK
_GRID = _NTR + _NCOMP


def _body(w0_ref, w1_ref, tok_ref, y_ref, td_ref):
    i = pl.program_id(0)

    @pl.when(i < _NTR)
    def _transpose_chunk():
        for bcl in range(_CHUNK // 128):
            src = tok_ref[pl.ds(bcl * 128, 128), :]
            lof = src[:, :128].T.astype(jnp.float32)
            hif = src[:, 128:].T[:72, :].astype(jnp.float32)
            col = 2 * (i * (_CHUNK // 128) + bcl)
            lof2 = jnp.broadcast_to(lof[:, None, :], (128, 2, 128))
            hif2 = jnp.broadcast_to(hif[:, None, :], (72, 2, 128))
            td_ref[pl.ds(0, 128), pl.ds(col, 2), :] = lof2
            td_ref[pl.ds(128, 72), pl.ds(col, 2), :] = hif2

    @pl.when(i >= _NTR)
    def _compute():
        w0 = w0_ref[0]
        w1 = w1_ref[0]
        even = (jax.lax.broadcasted_iota(jnp.int32, (256, 128), 0) & 1) == 0
        r0 = (i - _NTR) * _KH
        for h in range(_KH):
            u = td_ref[pl.ds(r0 + h, 1)][0]
            y_ref[pl.ds(h * 256, 256), :] = jnp.where(even, w0 - w1 * u * u, -u)


def kernel(tokens, w0, w1):
    y = pl.pallas_call(
        _body,
        grid=(_GRID,),
        in_specs=[
            pl.BlockSpec(memory_space=pltpu.SMEM),
            pl.BlockSpec(memory_space=pltpu.SMEM),
            pl.BlockSpec((_CHUNK, _HIST), lambda i: (jnp.minimum(i, _NTR - 1), 0)),
        ],
        out_specs=pl.BlockSpec(
            (_KH * 256, 128), lambda i: (jnp.maximum(i - _NTR, 0), 0)
        ),
        out_shape=jax.ShapeDtypeStruct((2 * _HIST * _BT, 128), jnp.float32),
        scratch_shapes=[pltpu.VMEM((_HIST, 2 * _BT, 128), jnp.float32)],
        compiler_params=pltpu.CompilerParams(
            dimension_semantics=("arbitrary",),
        ),
    )(w0.reshape(1), w1.reshape(1), tokens)
    y4 = y.reshape(_HIST, _BT, 2, 128)
    return jnp.transpose(y4, (1, 3, 0, 2)).reshape(_BATCH, _HIST, 2)
